# Initial kernel scaffold; baseline (speedup 1.0000x reference)
import jax
import jax.numpy as jnp
from jax.experimental import pallas as pl


def kernel(x, edge_index, h_mat_edge, delta_y, Wr, br, W1, b1, W2, b2, W3, b3, W4, b4, W5, b5):
    raise NotImplementedError("write your pallas kernel here")


# speedup vs baseline: 4.9272x; 4.9272x over previous
import functools

import jax
import jax.numpy as jnp
from jax import lax
from jax.experimental import pallas as pl
from jax.experimental.pallas import tpu as pltpu
from jax.experimental.pallas import tpu_sc as plsc

N = 100000
E = 3200000
HM = 4
HID = 16
OUT = 4
PK = 8
WP = 8

NW = 32
CH = 100
SUP = 8
GEDGES = SUP * CH
TOT_G = E // GEDGES
GPW = TOT_G // NW
STRIPE = 6256
N_PAD = 16 * STRIPE

_mesh = lambda: plsc.VectorSubcoreMesh(core_axis_name="c", subcore_axis_name="s")


def _leaky(t):
    return jnp.where(t > 0, t, 0.01 * t)


def _xr_body(x_ref, wr_ref, br_ref, o_ref):
    o_ref[:] = jnp.dot(x_ref[:], wr_ref[:],
                       preferred_element_type=jnp.float32) + br_ref[:]


def _xr_call(x, Wrp, brp):
    bn = 10000
    return pl.pallas_call(
        _xr_body,
        grid=(N // bn,),
        in_specs=[
            pl.BlockSpec((bn, 16), lambda i: (i, 0)),
            pl.BlockSpec((16, WP), lambda i: (0, 0)),
            pl.BlockSpec((1, WP), lambda i: (0, 0)),
        ],
        out_specs=pl.BlockSpec((bn, WP), lambda i: (i, 0)),
        out_shape=jax.ShapeDtypeStruct((N, WP), jnp.float32),
    )(x, Wrp, brp)


def _gather_body(xr_hbm, dst2_hbm, g_hbm, idx_v, rows_v, sem):
    cid = lax.axis_index("c")
    sid = lax.axis_index("s")
    wid = cid * 16 + sid
    g0 = wid * GPW

    def body(gi, carry):
        crow = (g0 + gi) * SUP
        pltpu.sync_copy(dst2_hbm.at[pl.ds(crow, SUP)], idx_v)
        cps = [
            pltpu.async_copy(xr_hbm.at[idx_v.at[k]], rows_v.at[k], sem)
            for k in range(SUP)
        ]
        for c in cps:
            c.wait()
        pltpu.sync_copy(rows_v, g_hbm.at[pl.ds(crow, SUP)])
        return carry

    lax.fori_loop(0, GPW, body, 0)


def _gather_call(xr, dst2):
    k = functools.partial(
        pl.kernel,
        out_type=jax.ShapeDtypeStruct((E // CH, CH, WP), jnp.float32),
        mesh=_mesh(),
        scratch_types=[
            pltpu.VMEM((SUP, CH), jnp.int32),
            pltpu.VMEM((SUP, CH, WP), jnp.float32),
            pltpu.SemaphoreType.DMA,
        ],
        compiler_params=pltpu.CompilerParams(use_tc_tiling_on_sc=False),
    )(_gather_body)
    return k(xr, dst2)


def _mlp_body(g_ref, hm_ref, dy_ref, w1a, w1b, w2, w3, w4, w5,
              b1r, b2r, b3r, b4r, b5r, rmat, o_ref):
    f32 = jnp.float32
    t = jnp.dot(g_ref[:], w1a[:], preferred_element_type=f32)
    t = t + jnp.dot(hm_ref[:], w1b[:], preferred_element_type=f32) + b1r[:]
    h = _leaky(t)
    h = _leaky(jnp.dot(h, w2[:], preferred_element_type=f32) + b2r[:])
    h = _leaky(jnp.dot(h, w3[:], preferred_element_type=f32) + b3r[:])
    h = _leaky(jnp.dot(h, w4[:], preferred_element_type=f32) + b4r[:])
    m = jnp.dot(h, w5[:], preferred_element_type=f32) + b5r[:]
    dyr = jnp.dot(dy_ref[:], rmat[:], preferred_element_type=f32)
    o_ref[:] = m * dyr


def _mlp_call(g8, hm8, dy8, w1a, w1b, w2p, w3p, w4p, w5p,
              b1r, b2r, b3r, b4r, b5r, rmat):
    br_ = 1600
    rows = E // PK
    lane = PK * HID
    wspec = lambda shape: pl.BlockSpec(shape, lambda i: (0, 0))
    return pl.pallas_call(
        _mlp_body,
        grid=(rows // br_,),
        in_specs=[
            pl.BlockSpec((br_, PK * WP), lambda i: (i, 0)),
            pl.BlockSpec((br_, PK * HM), lambda i: (i, 0)),
            pl.BlockSpec((br_, PK), lambda i: (i, 0)),
            wspec((PK * WP, lane)), wspec((PK * HM, lane)),
            wspec((lane, lane)), wspec((lane, lane)), wspec((lane, lane)),
            wspec((lane, PK * WP)),
            wspec((1, lane)), wspec((1, lane)), wspec((1, lane)),
            wspec((1, lane)), wspec((1, PK * WP)), wspec((PK, PK * WP)),
        ],
        out_specs=pl.BlockSpec((br_, PK * WP), lambda i: (i, 0)),
        out_shape=jax.ShapeDtypeStruct((rows, PK * WP), jnp.float32),
    )(g8, hm8, dy8, w1a, w1b, w2p, w3p, w4p, w5p,
      b1r, b2r, b3r, b4r, b5r, rmat)


def _scatter_body(m_hbm, dst2_hbm, z4_hbm, z1_hbm,
                  sums_hbm, cnts_hbm,
                  idx_v, rows_v, ones_v, ssum, scnt, sem1, sem2):
    cid = lax.axis_index("c")
    sid = lax.axis_index("s")
    wid = cid * 16 + sid
    g0 = wid * GPW

    for i in range(112 // 16):
        ones_v[pl.ds(i * 16, 16)] = jnp.ones((16,), jnp.float32)
    pltpu.sync_copy(z4_hbm, ssum.at[pl.ds(sid * STRIPE, STRIPE)])
    pltpu.sync_copy(z1_hbm, scnt.at[pl.ds(sid * STRIPE, STRIPE)])
    plsc.subcore_barrier()

    def body(gi, carry):
        crow = (g0 + gi) * SUP
        pltpu.sync_copy(dst2_hbm.at[pl.ds(crow, SUP)], idx_v)
        pltpu.sync_copy(m_hbm.at[pl.ds(crow, SUP)], rows_v)
        cps = []
        for k in range(SUP):
            cps.append(pltpu.async_copy(
                rows_v.at[k], ssum.at[idx_v.at[k]], sem1, add=True))
            cps.append(pltpu.async_copy(
                ones_v.at[pl.ds(0, CH)], scnt.at[idx_v.at[k]], sem2,
                add=True))
        for c in cps:
            c.wait()
        return carry

    lax.fori_loop(0, GPW, body, 0)
    plsc.subcore_barrier()

    sl = pl.ds(sid * STRIPE, STRIPE)
    pltpu.sync_copy(ssum.at[sl], sums_hbm.at[cid, sl])
    pltpu.sync_copy(scnt.at[sl], cnts_hbm.at[cid, sl])


def _scatter_call(m_flat, dst2, z4, z1):
    k = functools.partial(
        pl.kernel,
        out_type=(
            jax.ShapeDtypeStruct((2, N_PAD, WP), jnp.float32),
            jax.ShapeDtypeStruct((2, N_PAD), jnp.float32),
        ),
        mesh=_mesh(),
        scratch_types=[
            pltpu.VMEM((SUP, CH), jnp.int32),
            pltpu.VMEM((SUP, CH, WP), jnp.float32),
            pltpu.VMEM((112,), jnp.float32),
            pltpu.VMEM_SHARED((N_PAD, WP), jnp.float32),
            pltpu.VMEM_SHARED((N_PAD,), jnp.float32),
            pltpu.SemaphoreType.DMA,
            pltpu.SemaphoreType.DMA,
        ],
        compiler_params=pltpu.CompilerParams(use_tc_tiling_on_sc=False),
    )(_scatter_body)
    return k(m_flat, dst2, z4, z1)


def _comb_body(s_ref, c_ref, o_ref):
    cnt = c_ref[0] + c_ref[1]
    cnt = jnp.maximum(cnt, 1.0)
    o_ref[:] = (s_ref[0, :, :OUT] + s_ref[1, :, :OUT]) / cnt


def _comb_call(sums, cnts):
    bn = 10000
    return pl.pallas_call(
        _comb_body,
        grid=(N // bn,),
        in_specs=[
            pl.BlockSpec((2, bn, WP), lambda i: (0, i, 0)),
            pl.BlockSpec((2, bn, 1), lambda i: (0, i, 0)),
        ],
        out_specs=pl.BlockSpec((bn, OUT), lambda i: (i, 0)),
        out_shape=jax.ShapeDtypeStruct((N, OUT), jnp.float32),
    )(sums, cnts)


def kernel(x, edge_index, h_mat_edge, delta_y,
           Wr, br, W1, b1, W2, b2, W3, b3, W4, b4, W5, b5):
    f32 = jnp.float32
    dst = edge_index[1]
    dst2 = dst.reshape(E // CH, CH)

    xr = _xr_call(x, jnp.pad(Wr, ((0, 0), (0, WP - HM))),
                  jnp.pad(br, (0, WP - HM))[None, :])
    g = _gather_call(xr, dst2)

    eye = jnp.eye(PK, dtype=f32)
    w1a = jnp.kron(eye, jnp.pad(W1[:HM], ((0, WP - HM), (0, 0))))
    w1b = jnp.kron(eye, W1[HM:])
    w2p = jnp.kron(eye, W2)
    w3p = jnp.kron(eye, W3)
    w4p = jnp.kron(eye, W4)
    w5p = jnp.kron(eye, jnp.pad(W5, ((0, 0), (0, WP - OUT))))
    rmat = jnp.kron(eye, jnp.ones((1, WP), f32))
    b1r = jnp.tile(b1, PK)[None, :]
    b2r = jnp.tile(b2, PK)[None, :]
    b3r = jnp.tile(b3, PK)[None, :]
    b4r = jnp.tile(b4, PK)[None, :]
    b5r = jnp.tile(jnp.pad(b5, (0, WP - OUT)), PK)[None, :]

    m8 = _mlp_call(g.reshape(E // PK, PK * WP),
                   h_mat_edge.reshape(E // PK, PK * HM),
                   delta_y.reshape(E // PK, PK),
                   w1a, w1b, w2p, w3p, w4p, w5p,
                   b1r, b2r, b3r, b4r, b5r, rmat)

    z4 = jnp.zeros((STRIPE, WP), f32)
    z1 = jnp.zeros((STRIPE,), f32)
    sums, cnts = _scatter_call(m8.reshape(E // CH, CH, WP), dst2, z4, z1)

    return _comb_call(sums, cnts.reshape(2, N_PAD, 1))


# speedup vs baseline: 6.1181x; 1.2417x over previous
import functools

import jax
import jax.numpy as jnp
from jax import lax
from jax.experimental import pallas as pl
from jax.experimental.pallas import tpu as pltpu
from jax.experimental.pallas import tpu_sc as plsc

N = 100000
E = 3200000
HM = 4
HID = 16
OUT = 4
PK = 8
WP = 8

NW = 32
CH = 100
SUP = 8
GEDGES = SUP * CH
TOT_G = E // GEDGES
GPW = TOT_G // NW
STRIPE = 6256
N_PAD = 16 * STRIPE

_mesh = lambda: plsc.VectorSubcoreMesh(core_axis_name="c", subcore_axis_name="s")


def _leaky(t):
    return jnp.where(t > 0, t, 0.01 * t)


def _xr_body(x_ref, wr_ref, br_ref, o_ref):
    o_ref[:] = jnp.dot(x_ref[:], wr_ref[:],
                       preferred_element_type=jnp.float32) + br_ref[:]


def _xr_call(x, Wrp, brp):
    bn = 10000
    return pl.pallas_call(
        _xr_body,
        grid=(N // bn,),
        in_specs=[
            pl.BlockSpec((bn, 16), lambda i: (i, 0)),
            pl.BlockSpec((16, WP), lambda i: (0, 0)),
            pl.BlockSpec((1, WP), lambda i: (0, 0)),
        ],
        out_specs=pl.BlockSpec((bn, WP), lambda i: (i, 0)),
        out_shape=jax.ShapeDtypeStruct((N, WP), jnp.float32),
    )(x, Wrp, brp)


def _gather_body(xr_hbm, dst2_hbm, g_hbm, idx_v, rows_v, sem):
    cid = lax.axis_index("c")
    sid = lax.axis_index("s")
    wid = cid * 16 + sid
    g0 = wid * GPW

    def body(gi, carry):
        crow = (g0 + gi) * SUP
        pltpu.sync_copy(dst2_hbm.at[pl.ds(crow, SUP)], idx_v)
        cps = [
            pltpu.async_copy(xr_hbm.at[idx_v.at[k]], rows_v.at[k], sem)
            for k in range(SUP)
        ]
        for c in cps:
            c.wait()
        pltpu.sync_copy(rows_v, g_hbm.at[pl.ds(crow, SUP)])
        return carry

    lax.fori_loop(0, GPW, body, 0)


def _gather_call(xr, dst2):
    k = functools.partial(
        pl.kernel,
        out_type=jax.ShapeDtypeStruct((E // CH, CH, WP), jnp.float32),
        mesh=_mesh(),
        scratch_types=[
            pltpu.VMEM((SUP, CH), jnp.int32),
            pltpu.VMEM((SUP, CH, WP), jnp.float32),
            pltpu.SemaphoreType.DMA,
        ],
        compiler_params=pltpu.CompilerParams(use_tc_tiling_on_sc=False),
    )(_gather_body)
    return k(xr, dst2)


def _mlp_body(g_ref, hm_ref, dy_ref, w1a, w1b, w2, w3, w4, w5,
              b1c, b2c, b3c, b4c, b5c, o_ref):
    f32 = jnp.float32
    gt = jnp.transpose(g_ref[:])
    t = jnp.dot(w1a[:], gt, preferred_element_type=f32)
    t = t + jnp.dot(w1b[:], hm_ref[:], preferred_element_type=f32) + b1c[:]
    h = _leaky(t)
    h = _leaky(jnp.dot(w2[:], h, preferred_element_type=f32) + b2c[:])
    h = _leaky(jnp.dot(w3[:], h, preferred_element_type=f32) + b3c[:])
    h = _leaky(jnp.dot(w4[:], h, preferred_element_type=f32) + b4c[:])
    mt = jnp.dot(w5[:], h, preferred_element_type=f32) + b5c[:]
    mt = mt * dy_ref[:]
    o_ref[:] = jnp.transpose(mt)


def _mlp_call(g_em, hmt, dyt, w1a, w1b, w2t, w3t, w4t, w5t,
              b1c, b2c, b3c, b4c, b5c):
    be = 12800
    wspec = lambda shape: pl.BlockSpec(shape, lambda i: (0, 0))
    return pl.pallas_call(
        _mlp_body,
        grid=(E // be,),
        in_specs=[
            pl.BlockSpec((be, WP), lambda i: (i, 0)),
            pl.BlockSpec((HM, be), lambda i: (0, i)),
            pl.BlockSpec((1, be), lambda i: (0, i)),
            wspec((HID, WP)), wspec((HID, HM)),
            wspec((HID, HID)), wspec((HID, HID)), wspec((HID, HID)),
            wspec((WP, HID)),
            wspec((HID, 1)), wspec((HID, 1)), wspec((HID, 1)),
            wspec((HID, 1)), wspec((WP, 1)),
        ],
        out_specs=pl.BlockSpec((be, WP), lambda i: (i, 0)),
        out_shape=jax.ShapeDtypeStruct((E, WP), jnp.float32),
    )(g_em, hmt, dyt, w1a, w1b, w2t, w3t, w4t, w5t,
      b1c, b2c, b3c, b4c, b5c)


def _scatter_body(m_hbm, dst2_hbm, z4_hbm, z1_hbm,
                  sums_hbm, cnts_hbm,
                  idx_v, rows_v, ones_v, ssum, scnt, sem1, sem2):
    cid = lax.axis_index("c")
    sid = lax.axis_index("s")
    wid = cid * 16 + sid
    g0 = wid * GPW

    for i in range(112 // 16):
        ones_v[pl.ds(i * 16, 16)] = jnp.ones((16,), jnp.float32)
    pltpu.sync_copy(z4_hbm, ssum.at[pl.ds(sid * STRIPE, STRIPE)])
    pltpu.sync_copy(z1_hbm, scnt.at[pl.ds(sid * STRIPE, STRIPE)])
    plsc.subcore_barrier()

    def body(gi, carry):
        crow = (g0 + gi) * SUP
        pltpu.sync_copy(dst2_hbm.at[pl.ds(crow, SUP)], idx_v)
        pltpu.sync_copy(m_hbm.at[pl.ds(crow, SUP)], rows_v)
        cps = []
        for k in range(SUP):
            cps.append(pltpu.async_copy(
                rows_v.at[k], ssum.at[idx_v.at[k]], sem1, add=True))
            cps.append(pltpu.async_copy(
                ones_v.at[pl.ds(0, CH)], scnt.at[idx_v.at[k]], sem2,
                add=True))
        for c in cps:
            c.wait()
        return carry

    lax.fori_loop(0, GPW, body, 0)
    plsc.subcore_barrier()

    sl = pl.ds(sid * STRIPE, STRIPE)
    pltpu.sync_copy(ssum.at[sl], sums_hbm.at[cid, sl])
    pltpu.sync_copy(scnt.at[sl], cnts_hbm.at[cid, sl])


def _scatter_call(m_flat, dst2, z4, z1):
    k = functools.partial(
        pl.kernel,
        out_type=(
            jax.ShapeDtypeStruct((2, N_PAD, WP), jnp.float32),
            jax.ShapeDtypeStruct((2, N_PAD), jnp.float32),
        ),
        mesh=_mesh(),
        scratch_types=[
            pltpu.VMEM((SUP, CH), jnp.int32),
            pltpu.VMEM((SUP, CH, WP), jnp.float32),
            pltpu.VMEM((112,), jnp.float32),
            pltpu.VMEM_SHARED((N_PAD, WP), jnp.float32),
            pltpu.VMEM_SHARED((N_PAD,), jnp.float32),
            pltpu.SemaphoreType.DMA,
            pltpu.SemaphoreType.DMA,
        ],
        compiler_params=pltpu.CompilerParams(use_tc_tiling_on_sc=False),
    )(_scatter_body)
    return k(m_flat, dst2, z4, z1)


def _comb_body(s_ref, c_ref, o_ref):
    cnt = c_ref[0] + c_ref[1]
    cnt = jnp.maximum(cnt, 1.0)
    o_ref[:] = (s_ref[0, :, :OUT] + s_ref[1, :, :OUT]) / cnt


def _comb_call(sums, cnts):
    bn = 10000
    return pl.pallas_call(
        _comb_body,
        grid=(N // bn,),
        in_specs=[
            pl.BlockSpec((2, bn, WP), lambda i: (0, i, 0)),
            pl.BlockSpec((2, bn, 1), lambda i: (0, i, 0)),
        ],
        out_specs=pl.BlockSpec((bn, OUT), lambda i: (i, 0)),
        out_shape=jax.ShapeDtypeStruct((N, OUT), jnp.float32),
    )(sums, cnts)


def kernel(x, edge_index, h_mat_edge, delta_y,
           Wr, br, W1, b1, W2, b2, W3, b3, W4, b4, W5, b5):
    f32 = jnp.float32
    dst = edge_index[1]
    dst2 = dst.reshape(E // CH, CH)

    xr = _xr_call(x, jnp.pad(Wr, ((0, 0), (0, WP - HM))),
                  jnp.pad(br, (0, WP - HM))[None, :])
    g = _gather_call(xr, dst2)

    w1a = jnp.pad(W1[:HM], ((0, WP - HM), (0, 0))).T
    w1b = W1[HM:].T
    w5t = jnp.pad(W5, ((0, 0), (0, WP - OUT))).T
    b5c = jnp.pad(b5, (0, WP - OUT))[:, None]

    m_em = _mlp_call(g.reshape(E, WP),
                     h_mat_edge.T, delta_y.T,
                     w1a, w1b, W2.T, W3.T, W4.T, w5t,
                     b1[:, None], b2[:, None], b3[:, None], b4[:, None],
                     b5c)

    z4 = jnp.zeros((STRIPE, WP), f32)
    z1 = jnp.zeros((STRIPE,), f32)
    sums, cnts = _scatter_call(m_em.reshape(E // CH, CH, WP), dst2, z4, z1)

    return _comb_call(sums, cnts.reshape(2, N_PAD, 1))


# speedup vs baseline: 6.8125x; 1.1135x over previous
import functools

import jax
import jax.numpy as jnp
from jax import lax
from jax.experimental import pallas as pl
from jax.experimental.pallas import tpu as pltpu
from jax.experimental.pallas import tpu_sc as plsc

N = 100000
E = 3200000
HM = 4
HID = 16
OUT = 4
PK = 8
WP = 8

NW = 32
CH = 100
SUP = 8
GEDGES = SUP * CH
TOT_G = E // GEDGES
GPW = TOT_G // NW
STRIPE = 6256
N_PAD = 16 * STRIPE

_mesh = lambda: plsc.VectorSubcoreMesh(core_axis_name="c", subcore_axis_name="s")


def _leaky(t):
    return jnp.where(t > 0, t, 0.01 * t)


def _xr_body(x_ref, wr_ref, br_ref, o_ref):
    o_ref[:] = jnp.dot(x_ref[:], wr_ref[:],
                       preferred_element_type=jnp.float32) + br_ref[:]


def _xr_call(x, Wrp, brp):
    bn = 10000
    return pl.pallas_call(
        _xr_body,
        grid=(N // bn,),
        in_specs=[
            pl.BlockSpec((bn, 16), lambda i: (i, 0)),
            pl.BlockSpec((16, WP), lambda i: (0, 0)),
            pl.BlockSpec((1, WP), lambda i: (0, 0)),
        ],
        out_specs=pl.BlockSpec((bn, WP), lambda i: (i, 0)),
        out_shape=jax.ShapeDtypeStruct((N, WP), jnp.float32),
    )(x, Wrp, brp)


def _gather_body(xr_hbm, dst2_hbm, g_hbm, idx_v, rows_v, sem):
    cid = lax.axis_index("c")
    sid = lax.axis_index("s")
    wid = cid * 16 + sid
    g0 = wid * GPW

    def body(gi, carry):
        crow = (g0 + gi) * SUP
        pltpu.sync_copy(dst2_hbm.at[pl.ds(crow, SUP)], idx_v)
        cps = [
            pltpu.async_copy(xr_hbm.at[idx_v.at[k]], rows_v.at[k], sem)
            for k in range(SUP)
        ]
        for c in cps:
            c.wait()
        pltpu.sync_copy(rows_v, g_hbm.at[pl.ds(crow, SUP)])
        return carry

    lax.fori_loop(0, GPW, body, 0)


def _gather_call(xr, dst2):
    k = functools.partial(
        pl.kernel,
        out_type=jax.ShapeDtypeStruct((E // CH, CH, WP), jnp.float32),
        mesh=_mesh(),
        scratch_types=[
            pltpu.VMEM((SUP, CH), jnp.int32),
            pltpu.VMEM((SUP, CH, WP), jnp.float32),
            pltpu.SemaphoreType.DMA,
        ],
        compiler_params=pltpu.CompilerParams(use_tc_tiling_on_sc=False),
    )(_gather_body)
    return k(xr, dst2)


def _mlp_body(g_ref, hm_ref, dy_ref, w1a, w1b, w2, w3, w4, w5,
              b1c, b2c, b3c, b4c, b5c, o_ref):
    f32 = jnp.float32
    gt = jnp.transpose(g_ref[:].reshape(BE, WP))
    t = jnp.dot(w1a[:], gt, preferred_element_type=f32)
    t = t + jnp.dot(w1b[:], hm_ref[:], preferred_element_type=f32) + b1c[:]
    h = _leaky(t)
    h = _leaky(jnp.dot(w2[:], h, preferred_element_type=f32) + b2c[:])
    h = _leaky(jnp.dot(w3[:], h, preferred_element_type=f32) + b3c[:])
    h = _leaky(jnp.dot(w4[:], h, preferred_element_type=f32) + b4c[:])
    mt = jnp.dot(w5[:], h, preferred_element_type=f32) + b5c[:]
    mt = mt * dy_ref[:]
    o_ref[:] = jnp.transpose(mt).reshape(BE // CH, CH, WP)


BE = 12800


def _mlp_call(g3, hmt, dyt, w1a, w1b, w2t, w3t, w4t, w5t,
              b1c, b2c, b3c, b4c, b5c):
    bch = BE // CH
    wspec = lambda shape: pl.BlockSpec(shape, lambda i: (0, 0))
    return pl.pallas_call(
        _mlp_body,
        grid=(E // BE,),
        in_specs=[
            pl.BlockSpec((bch, CH, WP), lambda i: (i, 0, 0)),
            pl.BlockSpec((HM, BE), lambda i: (0, i)),
            pl.BlockSpec((1, BE), lambda i: (0, i)),
            wspec((HID, WP)), wspec((HID, HM)),
            wspec((HID, HID)), wspec((HID, HID)), wspec((HID, HID)),
            wspec((WP, HID)),
            wspec((HID, 1)), wspec((HID, 1)), wspec((HID, 1)),
            wspec((HID, 1)), wspec((WP, 1)),
        ],
        out_specs=pl.BlockSpec((bch, CH, WP), lambda i: (i, 0, 0)),
        out_shape=jax.ShapeDtypeStruct((E // CH, CH, WP), jnp.float32),
    )(g3, hmt, dyt, w1a, w1b, w2t, w3t, w4t, w5t,
      b1c, b2c, b3c, b4c, b5c)


def _scatter_body(m_hbm, dst2_hbm, z4_hbm, z1_hbm,
                  sums_hbm, cnts_hbm,
                  idx_v, rows_v, ones_v, ssum, scnt, sem1, sem2):
    cid = lax.axis_index("c")
    sid = lax.axis_index("s")
    wid = cid * 16 + sid
    g0 = wid * GPW

    for i in range(112 // 16):
        ones_v[pl.ds(i * 16, 16)] = jnp.ones((16,), jnp.float32)
    pltpu.sync_copy(z4_hbm, ssum.at[pl.ds(sid * STRIPE, STRIPE)])
    pltpu.sync_copy(z1_hbm, scnt.at[pl.ds(sid * STRIPE, STRIPE)])
    plsc.subcore_barrier()

    def body(gi, carry):
        crow = (g0 + gi) * SUP
        pltpu.sync_copy(dst2_hbm.at[pl.ds(crow, SUP)], idx_v)
        pltpu.sync_copy(m_hbm.at[pl.ds(crow, SUP)], rows_v)
        cps = []
        for k in range(SUP):
            cps.append(pltpu.async_copy(
                rows_v.at[k], ssum.at[idx_v.at[k]], sem1, add=True))
            cps.append(pltpu.async_copy(
                ones_v.at[pl.ds(0, CH)], scnt.at[idx_v.at[k]], sem2,
                add=True))
        for c in cps:
            c.wait()
        return carry

    lax.fori_loop(0, GPW, body, 0)
    plsc.subcore_barrier()

    sl = pl.ds(sid * STRIPE, STRIPE)
    pltpu.sync_copy(ssum.at[sl], sums_hbm.at[cid, sl])
    pltpu.sync_copy(scnt.at[sl], cnts_hbm.at[cid, sl])


def _scatter_call(m_flat, dst2, z4, z1):
    k = functools.partial(
        pl.kernel,
        out_type=(
            jax.ShapeDtypeStruct((2, N_PAD, WP), jnp.float32),
            jax.ShapeDtypeStruct((2, N_PAD), jnp.float32),
        ),
        mesh=_mesh(),
        scratch_types=[
            pltpu.VMEM((SUP, CH), jnp.int32),
            pltpu.VMEM((SUP, CH, WP), jnp.float32),
            pltpu.VMEM((112,), jnp.float32),
            pltpu.VMEM_SHARED((N_PAD, WP), jnp.float32),
            pltpu.VMEM_SHARED((N_PAD,), jnp.float32),
            pltpu.SemaphoreType.DMA,
            pltpu.SemaphoreType.DMA,
        ],
        compiler_params=pltpu.CompilerParams(use_tc_tiling_on_sc=False),
    )(_scatter_body)
    return k(m_flat, dst2, z4, z1)


def _comb_body(s_ref, c_ref, o_ref):
    cnt = c_ref[0] + c_ref[1]
    cnt = jnp.maximum(cnt, 1.0)
    o_ref[:] = (s_ref[0, :, :OUT] + s_ref[1, :, :OUT]) / cnt


def _comb_call(sums, cnts):
    bn = 10000
    return pl.pallas_call(
        _comb_body,
        grid=(N // bn,),
        in_specs=[
            pl.BlockSpec((2, bn, WP), lambda i: (0, i, 0)),
            pl.BlockSpec((2, bn, 1), lambda i: (0, i, 0)),
        ],
        out_specs=pl.BlockSpec((bn, OUT), lambda i: (i, 0)),
        out_shape=jax.ShapeDtypeStruct((N, OUT), jnp.float32),
    )(sums, cnts)


def kernel(x, edge_index, h_mat_edge, delta_y,
           Wr, br, W1, b1, W2, b2, W3, b3, W4, b4, W5, b5):
    f32 = jnp.float32
    dst = edge_index[1]
    dst2 = dst.reshape(E // CH, CH)

    xr = _xr_call(x, jnp.pad(Wr, ((0, 0), (0, WP - HM))),
                  jnp.pad(br, (0, WP - HM))[None, :])
    g = _gather_call(xr, dst2)

    w1a = jnp.pad(W1[:HM], ((0, WP - HM), (0, 0))).T
    w1b = W1[HM:].T
    w5t = jnp.pad(W5, ((0, 0), (0, WP - OUT))).T
    b5c = jnp.pad(b5, (0, WP - OUT))[:, None]

    m3 = _mlp_call(g,
                   h_mat_edge.T, delta_y.T,
                   w1a, w1b, W2.T, W3.T, W4.T, w5t,
                   b1[:, None], b2[:, None], b3[:, None], b4[:, None],
                   b5c)

    z4 = jnp.zeros((STRIPE, WP), f32)
    z1 = jnp.zeros((STRIPE,), f32)
    sums, cnts = _scatter_call(m3, dst2, z4, z1)

    return _comb_call(sums, cnts.reshape(2, N_PAD, 1))
